# Initial kernel scaffold; baseline (speedup 1.0000x reference)
import jax
import jax.numpy as jnp
from jax.experimental import pallas as pl


def kernel(x):
    raise NotImplementedError("write your pallas kernel here")


# speedup vs baseline: 22579.9317x; 22579.9317x over previous
import functools

import jax
import jax.numpy as jnp
import numpy as np
from jax.experimental import pallas as pl
from jax.experimental.pallas import tpu as pltpu

N_BINS = 64
GRID = (4, 4, 4)
BANDWIDTH = 1e-3
CLIP_LIMIT = 4.0
D = H = W = 64
VOXELS = (D // GRID[0]) * (H // GRID[1]) * (W // GRID[2])
LIMIT = max(CLIP_LIMIT * VOXELS // N_BINS, 1.0)


def _b5_np(t):
    x = np.abs(t)
    p3 = np.clip(3.0 - x, 0.0, None) ** 5
    p2 = np.clip(2.0 - x, 0.0, None) ** 5
    p1 = np.clip(1.0 - x, 0.0, None) ** 5
    return (p3 - 6.0 * p2 + 15.0 * p1) / 120.0


def _fold_np(i, n):
    j = np.mod(i, 2 * n)
    return np.where(j >= n, 2 * n - 1 - j, j)


@functools.lru_cache(maxsize=None)
def _spatial_consts():
    g = GRID[0]
    c = np.linspace(-0.5 - 0.25 / g, g - 1 + 0.5 + 0.25 / g, D)
    base = np.floor(c)
    offs = np.arange(-2, 4)
    idx = base[:, None] + offs[None, :]
    wts = _b5_np(c[:, None] - idx)
    fold = _fold_np(idx.astype(np.int64), g)
    M = np.zeros((D, g))
    for t in range(6):
        np.add.at(M, (np.arange(D), fold[:, t]), wts[:, t])
    KL = np.einsum('hk,wl->klhw', M, M).reshape(g * g, H * W)
    return (jnp.asarray(M, dtype=jnp.float32),
            jnp.asarray(KL, dtype=jnp.float32))


def _hist_kernel(tiles_ref, cdf_ref):
    v = tiles_ref[0]
    bins_col = jax.lax.broadcasted_iota(jnp.int32, (N_BINS, 1), 0).astype(
        jnp.float32) * (1.0 / (N_BINS - 1))
    wts = jnp.exp(-0.5 * ((v - bins_col) * (1.0 / BANDWIDTH)) ** 2)
    pdf = jnp.sum(wts, axis=1, keepdims=True) * (1.0 / VOXELS)
    pdf = pdf / (jnp.sum(pdf) + 1e-10)
    histos = jnp.minimum(pdf * VOXELS, LIMIT)
    clipped = VOXELS - jnp.sum(histos)
    redist = jnp.floor(clipped * (1.0 / N_BINS))
    residual = clipped - redist * N_BINS
    iota = jax.lax.broadcasted_iota(jnp.int32, (N_BINS, 1), 0).astype(
        jnp.float32)
    histos = histos + redist + (iota < residual).astype(jnp.float32)
    r = jax.lax.broadcasted_iota(jnp.int32, (N_BINS, N_BINS), 0)
    cmat = jax.lax.broadcasted_iota(jnp.int32, (N_BINS, N_BINS), 1)
    ltri = (r >= cmat).astype(jnp.float32)
    cdf = jax.lax.dot_general(ltri, histos, (((1,), (0,)), ((), ())),
                              preferred_element_type=jnp.float32)
    cdf_ref[0] = cdf * ((N_BINS - 1.0) / VOXELS)


def _b5_jnp(t):
    x = jnp.abs(t)
    a3 = jnp.maximum(3.0 - x, 0.0)
    a2 = jnp.maximum(2.0 - x, 0.0)
    a1 = jnp.maximum(1.0 - x, 0.0)
    p3 = (a3 * a3) * (a3 * a3) * a3
    p2 = (a2 * a2) * (a2 * a2) * a2
    p1 = (a1 * a1) * (a1 * a1) * a1
    return (p3 - 6.0 * p2 + 15.0 * p1) * (1.0 / 120.0)


def _apply_kernel(m_ref, x_ref, cdf_ref, kl_ref, out_ref, mn_ref, mx_ref):
    d = pl.program_id(0)
    p0 = x_ref[0] * (N_BINS - 1.0)
    j_col = jax.lax.broadcasted_iota(jnp.int32, (N_BINS, 1), 0).astype(
        jnp.float32)
    w0t = _b5_jnp(p0 - j_col)
    i_col = jax.lax.broadcasted_iota(jnp.int32, (N_BINS, 1), 0)
    w0t += (i_col == 0).astype(jnp.float32) * _b5_jnp(p0 + 1.0)
    w0t += (i_col == 1).astype(jnp.float32) * _b5_jnp(p0 + 2.0)
    w0t += (i_col == N_BINS - 1).astype(jnp.float32) * _b5_jnp(p0 - N_BINS)
    w0t += (i_col == N_BINS - 2).astype(jnp.float32) * _b5_jnp(
        p0 - (N_BINS + 1.0))
    a_t = jax.lax.dot_general(cdf_ref[...], w0t, (((0,), (0,)), ((), ())),
                              preferred_element_type=jnp.float32)
    kl = kl_ref[...]
    g_t = jnp.concatenate(
        [m_ref[d, i] * kl for i in range(GRID[0])], axis=0)
    row = jnp.sum(a_t * g_t, axis=0, keepdims=True)
    out_ref[0] = row
    mn_ref[0] = jnp.broadcast_to(jnp.min(row), (1, 128))
    mx_ref[0] = jnp.broadcast_to(jnp.max(row), (1, 128))


def _norm_kernel(x_ref, mn_ref, mx_ref, out_ref):
    gmin = jnp.min(mn_ref[...])
    gmax = jnp.max(mx_ref[...])
    out_ref[0] = (x_ref[0] - gmin) / (gmax - gmin + 1e-10)


def kernel(x):
    gd, gh, gw = GRID
    td, th, tw = D // gd, H // gh, W // gw
    n_tiles = gd * gh * gw
    xf = x.reshape(D, H, W)
    tiles = xf.reshape(gd, td, gh, th, gw, tw).transpose(
        0, 2, 4, 1, 3, 5).reshape(n_tiles, 1, VOXELS)
    M, KL = _spatial_consts()

    cdf3 = pl.pallas_call(
        _hist_kernel,
        grid=(n_tiles,),
        in_specs=[pl.BlockSpec((1, 1, VOXELS), lambda t: (t, 0, 0))],
        out_specs=pl.BlockSpec((1, N_BINS, 1), lambda t: (t, 0, 0)),
        out_shape=jax.ShapeDtypeStruct((n_tiles, N_BINS, 1), jnp.float32),
    )(tiles)
    cdf = cdf3.reshape(n_tiles, N_BINS).T

    slices = xf.reshape(D, 1, H * W)
    unnorm, mins, maxs = pl.pallas_call(
        _apply_kernel,
        grid=(D,),
        in_specs=[
            pl.BlockSpec(memory_space=pltpu.SMEM),
            pl.BlockSpec((1, 1, H * W), lambda d: (d, 0, 0)),
            pl.BlockSpec((N_BINS, n_tiles), lambda d: (0, 0)),
            pl.BlockSpec((gh * gw, H * W), lambda d: (0, 0)),
        ],
        out_specs=[
            pl.BlockSpec((1, 1, H * W), lambda d: (d, 0, 0)),
            pl.BlockSpec((1, 1, 128), lambda d: (d, 0, 0)),
            pl.BlockSpec((1, 1, 128), lambda d: (d, 0, 0)),
        ],
        out_shape=[
            jax.ShapeDtypeStruct((D, 1, H * W), jnp.float32),
            jax.ShapeDtypeStruct((D, 1, 128), jnp.float32),
            jax.ShapeDtypeStruct((D, 1, 128), jnp.float32),
        ],
    )(M, slices, cdf, KL)

    out = pl.pallas_call(
        _norm_kernel,
        grid=(D,),
        in_specs=[
            pl.BlockSpec((1, 1, H * W), lambda d: (d, 0, 0)),
            pl.BlockSpec((D, 1, 128), lambda d: (0, 0, 0)),
            pl.BlockSpec((D, 1, 128), lambda d: (0, 0, 0)),
        ],
        out_specs=pl.BlockSpec((1, 1, H * W), lambda d: (d, 0, 0)),
        out_shape=jax.ShapeDtypeStruct((D, 1, H * W), jnp.float32),
    )(unnorm, mins, maxs)

    return out.reshape(1, 1, D, H, W)


# speedup vs baseline: 23628.1406x; 1.0464x over previous
import functools

import jax
import jax.numpy as jnp
import numpy as np
from jax.experimental import pallas as pl
from jax.experimental.pallas import tpu as pltpu

N_BINS = 64
GRID = (4, 4, 4)
BANDWIDTH = 1e-3
CLIP_LIMIT = 4.0
D = H = W = 64
VOXELS = (D // GRID[0]) * (H // GRID[1]) * (W // GRID[2])
LIMIT = max(CLIP_LIMIT * VOXELS // N_BINS, 1.0)
NEXT = N_BINS + 8
N_TILES = GRID[0] * GRID[1] * GRID[2]
HW = H * W


def _b5_np(t):
    x = np.abs(t)
    p3 = np.clip(3.0 - x, 0.0, None) ** 5
    p2 = np.clip(2.0 - x, 0.0, None) ** 5
    p1 = np.clip(1.0 - x, 0.0, None) ** 5
    return (p3 - 6.0 * p2 + 15.0 * p1) / 120.0


def _fold_np(i, n):
    j = np.mod(i, 2 * n)
    return np.where(j >= n, 2 * n - 1 - j, j)


@functools.lru_cache(maxsize=None)
def _spatial_consts():
    g = GRID[0]
    c = np.linspace(-0.5 - 0.25 / g, g - 1 + 0.5 + 0.25 / g, D)
    base = np.floor(c)
    offs = np.arange(-2, 4)
    idx = base[:, None] + offs[None, :]
    wts = _b5_np(c[:, None] - idx)
    fold = _fold_np(idx.astype(np.int64), g)
    M = np.zeros((D, g))
    for t in range(6):
        np.add.at(M, (np.arange(D), fold[:, t]), wts[:, t])
    KL = np.einsum('hk,wl->klhw', M, M).reshape(g * g, HW)
    return (jnp.asarray(M, dtype=jnp.float32),
            jnp.asarray(KL, dtype=jnp.float32))


def _b5_jnp(t):
    x = jnp.abs(t)
    a3 = jnp.maximum(3.0 - x, 0.0)
    a2 = jnp.maximum(2.0 - x, 0.0)
    a1 = jnp.maximum(1.0 - x, 0.0)
    p3 = (a3 * a3) * (a3 * a3) * a3
    p2 = (a2 * a2) * (a2 * a2) * a2
    p1 = (a1 * a1) * (a1 * a1) * a1
    return (p3 - 6.0 * p2 + 15.0 * p1) * (1.0 / 120.0)


def _fused_kernel(m_ref, tiles_ref, slices_ref, kl_ref, out_ref,
                  cdf_s, u_s, mm_s):
    p = pl.program_id(0)
    d = pl.program_id(1)

    @pl.when(jnp.logical_and(p == 0, d == 0))
    def _init():
        cdf_s[...] = jnp.zeros((NEXT, N_TILES), jnp.float32)
        mm_s[0] = jnp.float32(jnp.inf)
        mm_s[1] = jnp.float32(-jnp.inf)

    @pl.when(p == 0)
    def _hist():
        v = tiles_ref[0]
        bins_col = jax.lax.broadcasted_iota(
            jnp.int32, (N_BINS, 1), 0).astype(jnp.float32) * (
                1.0 / (N_BINS - 1))
        wts = jnp.exp(-0.5 * ((v - bins_col) * (1.0 / BANDWIDTH)) ** 2)
        pdf = jnp.sum(wts, axis=1, keepdims=True) * (1.0 / VOXELS)
        pdf = pdf / (jnp.sum(pdf) + 1e-10)
        histos = jnp.minimum(pdf * VOXELS, LIMIT)
        clipped = VOXELS - jnp.sum(histos)
        redist = jnp.floor(clipped * (1.0 / N_BINS))
        residual = clipped - redist * N_BINS
        iota = jax.lax.broadcasted_iota(
            jnp.int32, (N_BINS, 1), 0).astype(jnp.float32)
        histos = histos + redist + (iota < residual).astype(jnp.float32)
        r = jax.lax.broadcasted_iota(jnp.int32, (N_BINS, N_BINS), 0)
        cmat = jax.lax.broadcasted_iota(jnp.int32, (N_BINS, N_BINS), 1)
        ltri = (r >= cmat).astype(jnp.float32)
        cdf = jax.lax.dot_general(ltri, histos, (((1,), (0,)), ((), ())),
                                  preferred_element_type=jnp.float32)
        cdf = cdf * ((N_BINS - 1.0) / VOXELS)
        ext = jnp.concatenate(
            [cdf[3:4], cdf[2:3], cdf[1:2], cdf[0:1], cdf,
             cdf[63:64], cdf[62:63], cdf[61:62], cdf[60:61]], axis=0)
        lane = jax.lax.broadcasted_iota(jnp.int32, (NEXT, N_TILES), 1)
        cdf_s[...] += ext * (lane == d).astype(jnp.float32)

    @pl.when(p == 1)
    def _apply():
        p0 = slices_ref[0] * (N_BINS - 1.0)
        jp_col = jax.lax.broadcasted_iota(
            jnp.int32, (NEXT, 1), 0).astype(jnp.float32) - 4.0
        w0t = _b5_jnp(p0 - jp_col)
        a_t = jax.lax.dot_general(
            cdf_s[...], w0t, (((0,), (0,)), ((), ())),
            preferred_element_type=jnp.float32)
        kl = kl_ref[...]
        g_t = jnp.concatenate(
            [m_ref[d, i] * kl for i in range(GRID[0])], axis=0)
        row = jnp.sum(a_t * g_t, axis=0, keepdims=True)
        u_s[pl.ds(d, 1), :] = row
        mm_s[0] = jnp.minimum(mm_s[0], jnp.min(row))
        mm_s[1] = jnp.maximum(mm_s[1], jnp.max(row))

    @pl.when(p == 2)
    def _norm():
        gmin = mm_s[0]
        gmax = mm_s[1]
        out_ref[0] = (u_s[pl.ds(d, 1), :] - gmin) / (gmax - gmin + 1e-10)


def kernel(x):
    gd, gh, gw = GRID
    td, th, tw = D // gd, H // gh, W // gw
    xf = x.reshape(D, H, W)
    tiles = xf.reshape(gd, td, gh, th, gw, tw).transpose(
        0, 2, 4, 1, 3, 5).reshape(N_TILES, 1, VOXELS)
    slices = xf.reshape(D, 1, HW)
    M, KL = _spatial_consts()

    out = pl.pallas_call(
        _fused_kernel,
        grid=(3, D),
        in_specs=[
            pl.BlockSpec(memory_space=pltpu.SMEM),
            pl.BlockSpec((1, 1, VOXELS), lambda p, d: (d, 0, 0)),
            pl.BlockSpec((1, 1, HW), lambda p, d: (d, 0, 0)),
            pl.BlockSpec((gh * gw, HW), lambda p, d: (0, 0)),
        ],
        out_specs=pl.BlockSpec((1, 1, HW), lambda p, d: (d, 0, 0)),
        out_shape=jax.ShapeDtypeStruct((D, 1, HW), jnp.float32),
        scratch_shapes=[
            pltpu.VMEM((NEXT, N_TILES), jnp.float32),
            pltpu.VMEM((D, HW), jnp.float32),
            pltpu.SMEM((2,), jnp.float32),
        ],
    )(M, tiles, slices, KL)

    return out.reshape(1, 1, D, H, W)


# speedup vs baseline: 33926.6155x; 1.4359x over previous
import functools

import jax
import jax.numpy as jnp
import numpy as np
from jax.experimental import pallas as pl
from jax.experimental.pallas import tpu as pltpu

N_BINS = 64
GRID = (4, 4, 4)
BANDWIDTH = 1e-3
CLIP_LIMIT = 4.0
D = H = W = 64
VOXELS = (D // GRID[0]) * (H // GRID[1]) * (W // GRID[2])
LIMIT = max(CLIP_LIMIT * VOXELS // N_BINS, 1.0)
NEXT = N_BINS + 8
N_TILES = GRID[0] * GRID[1] * GRID[2]
HW = H * W
TB = 2
SB = 2
NB = 8
PH1 = N_TILES // TB
PH2 = PH1 + D // SB
PH3 = PH2 + D // NB


def _b5_np(t):
    x = np.abs(t)
    p3 = np.clip(3.0 - x, 0.0, None) ** 5
    p2 = np.clip(2.0 - x, 0.0, None) ** 5
    p1 = np.clip(1.0 - x, 0.0, None) ** 5
    return (p3 - 6.0 * p2 + 15.0 * p1) / 120.0


def _fold_np(i, n):
    j = np.mod(i, 2 * n)
    return np.where(j >= n, 2 * n - 1 - j, j)


@functools.lru_cache(maxsize=None)
def _spatial_consts():
    g = GRID[0]
    c = np.linspace(-0.5 - 0.25 / g, g - 1 + 0.5 + 0.25 / g, D)
    base = np.floor(c)
    offs = np.arange(-2, 4)
    idx = base[:, None] + offs[None, :]
    wts = _b5_np(c[:, None] - idx)
    fold = _fold_np(idx.astype(np.int64), g)
    M = np.zeros((D, g))
    for t in range(6):
        np.add.at(M, (np.arange(D), fold[:, t]), wts[:, t])
    KL = np.einsum('hk,wl->klhw', M, M).reshape(g * g, HW)
    return (jnp.asarray(M, dtype=jnp.float32),
            jnp.asarray(KL, dtype=jnp.float32))


def _b5_jnp(t):
    x = jnp.abs(t)
    a3 = jnp.maximum(3.0 - x, 0.0)
    a2 = jnp.maximum(2.0 - x, 0.0)
    a1 = jnp.maximum(1.0 - x, 0.0)
    p3 = (a3 * a3) * (a3 * a3) * a3
    p2 = (a2 * a2) * (a2 * a2) * a2
    p1 = (a1 * a1) * (a1 * a1) * a1
    return (p3 - 6.0 * p2 + 15.0 * p1) * (1.0 / 120.0)


def _fused_kernel(m_ref, tiles_ref, slices_ref, kl_ref, out_ref,
                  cdf_s, u_s, mm_s):
    g = pl.program_id(0)

    @pl.when(g == 0)
    def _init():
        cdf_s[...] = jnp.zeros((NEXT, N_TILES), jnp.float32)
        mm_s[0] = jnp.float32(jnp.inf)
        mm_s[1] = jnp.float32(-jnp.inf)

    @pl.when(g < PH1)
    def _hist():
        for r in range(TB):
            t_idx = g * TB + r
            v = tiles_ref[r]
            bins_col = jax.lax.broadcasted_iota(
                jnp.int32, (N_BINS, 1), 0).astype(jnp.float32) * (
                    1.0 / (N_BINS - 1))
            wts = jnp.exp(-0.5 * ((v - bins_col) * (1.0 / BANDWIDTH)) ** 2)
            pdf = jnp.sum(wts, axis=1, keepdims=True) * (1.0 / VOXELS)
            pdf = pdf / (jnp.sum(pdf) + 1e-10)
            histos = jnp.minimum(pdf * VOXELS, LIMIT)
            clipped = VOXELS - jnp.sum(histos)
            redist = jnp.floor(clipped * (1.0 / N_BINS))
            residual = clipped - redist * N_BINS
            iota = jax.lax.broadcasted_iota(
                jnp.int32, (N_BINS, 1), 0).astype(jnp.float32)
            histos = histos + redist + (iota < residual).astype(jnp.float32)
            rr = jax.lax.broadcasted_iota(jnp.int32, (N_BINS, N_BINS), 0)
            cc = jax.lax.broadcasted_iota(jnp.int32, (N_BINS, N_BINS), 1)
            ltri = (rr >= cc).astype(jnp.float32)
            cdf = jax.lax.dot_general(
                ltri, histos, (((1,), (0,)), ((), ())),
                preferred_element_type=jnp.float32)
            cdf = cdf * ((N_BINS - 1.0) / VOXELS)
            ext = jnp.concatenate(
                [cdf[3:4], cdf[2:3], cdf[1:2], cdf[0:1], cdf,
                 cdf[63:64], cdf[62:63], cdf[61:62], cdf[60:61]], axis=0)
            lane = jax.lax.broadcasted_iota(jnp.int32, (NEXT, N_TILES), 1)
            cdf_s[...] += ext * (lane == t_idx).astype(jnp.float32)

    @pl.when(jnp.logical_and(g >= PH1, g < PH2))
    def _apply():
        kl = kl_ref[...]
        for r in range(SB):
            d = (g - PH1) * SB + r
            p0 = slices_ref[r] * (N_BINS - 1.0)
            jp_col = jax.lax.broadcasted_iota(
                jnp.int32, (NEXT, 1), 0).astype(jnp.float32) - 4.0
            w0t = _b5_jnp(p0 - jp_col)
            c_d = (m_ref[d, 0] * cdf_s[:, 0:16]
                   + m_ref[d, 1] * cdf_s[:, 16:32]
                   + m_ref[d, 2] * cdf_s[:, 32:48]
                   + m_ref[d, 3] * cdf_s[:, 48:64])
            b = jax.lax.dot_general(
                c_d, w0t, (((0,), (0,)), ((), ())),
                preferred_element_type=jnp.float32)
            row = jnp.sum(b * kl, axis=0, keepdims=True)
            u_s[pl.ds(d, 1), :] = row
            mm_s[0] = jnp.minimum(mm_s[0], jnp.min(row))
            mm_s[1] = jnp.maximum(mm_s[1], jnp.max(row))

    @pl.when(g >= PH2)
    def _norm():
        d0 = (g - PH2) * NB
        gmin = mm_s[0]
        gmax = mm_s[1]
        out_ref[:, 0, :] = (u_s[pl.ds(d0, NB), :] - gmin) / (
            gmax - gmin + 1e-10)


def kernel(x):
    gd, gh, gw = GRID
    td, th, tw = D // gd, H // gh, W // gw
    xf = x.reshape(D, H, W)
    tiles = xf.reshape(gd, td, gh, th, gw, tw).transpose(
        0, 2, 4, 1, 3, 5).reshape(N_TILES, 1, VOXELS)
    slices = xf.reshape(D, 1, HW)
    M, KL = _spatial_consts()

    out = pl.pallas_call(
        _fused_kernel,
        grid=(PH3,),
        in_specs=[
            pl.BlockSpec(memory_space=pltpu.SMEM),
            pl.BlockSpec((TB, 1, VOXELS),
                         lambda g: (jnp.minimum(g, PH1 - 1), 0, 0)),
            pl.BlockSpec((SB, 1, HW),
                         lambda g: (jnp.clip(g - PH1, 0, D // SB - 1), 0, 0)),
            pl.BlockSpec((gh * gw, HW), lambda g: (0, 0)),
        ],
        out_specs=pl.BlockSpec((NB, 1, HW),
                               lambda g: (jnp.clip(g - PH2, 0,
                                                   D // NB - 1), 0, 0)),
        out_shape=jax.ShapeDtypeStruct((D, 1, HW), jnp.float32),
        scratch_shapes=[
            pltpu.VMEM((NEXT, N_TILES), jnp.float32),
            pltpu.VMEM((D, HW), jnp.float32),
            pltpu.SMEM((2,), jnp.float32),
        ],
    )(M, tiles, slices, KL)

    return out.reshape(1, 1, D, H, W)


# speedup vs baseline: 40197.8725x; 1.1848x over previous
import functools
import math

import jax
import jax.numpy as jnp
import numpy as np
from jax.experimental import pallas as pl
from jax.experimental.pallas import tpu as pltpu

N_BINS = 64
GRID = (4, 4, 4)
BANDWIDTH = 1e-3
CLIP_LIMIT = 4.0
D = H = W = 64
VOXELS = (D // GRID[0]) * (H // GRID[1]) * (W // GRID[2])
LIMIT = max(CLIP_LIMIT * VOXELS // N_BINS, 1.0)
NEXT = N_BINS + 8
N_TILES = GRID[0] * GRID[1] * GRID[2]
HW = H * W
TB = 2
SB = 2
NB = 8
PH1 = N_TILES // TB
PH2 = PH1 + D // SB
PH3 = PH2 + D // NB


def _b5_np(t):
    x = np.abs(t)
    p3 = np.clip(3.0 - x, 0.0, None) ** 5
    p2 = np.clip(2.0 - x, 0.0, None) ** 5
    p1 = np.clip(1.0 - x, 0.0, None) ** 5
    return (p3 - 6.0 * p2 + 15.0 * p1) / 120.0


def _fold_np(i, n):
    j = np.mod(i, 2 * n)
    return np.where(j >= n, 2 * n - 1 - j, j)


def _tap_coeffs():
    def pc(a, sign):
        return np.array([math.comb(5, k) * a ** (5 - k) * sign ** k
                         for k in range(6)], dtype=np.float64)
    Wc = np.zeros((6, 6))
    Wc[0] = pc(1, -1) / 120.0
    Wc[1] = (pc(2, -1) - 6 * pc(1, -1)) / 120.0
    Wc[2] = (pc(3, -1) - 6 * pc(2, -1) + 15 * pc(1, -1)) / 120.0
    Wc[3] = (pc(2, 1) - 6 * pc(1, 1) + 15 * pc(0, 1)) / 120.0
    Wc[4] = (pc(1, 1) - 6 * pc(0, 1)) / 120.0
    Wc[5] = pc(0, 1) / 120.0
    return Wc.astype(np.float32)


_WC = _tap_coeffs()


@functools.lru_cache(maxsize=None)
def _spatial_consts():
    g = GRID[0]
    c = np.linspace(-0.5 - 0.25 / g, g - 1 + 0.5 + 0.25 / g, D)
    base = np.floor(c)
    offs = np.arange(-2, 4)
    idx = base[:, None] + offs[None, :]
    wts = _b5_np(c[:, None] - idx)
    fold = _fold_np(idx.astype(np.int64), g)
    M = np.zeros((D, g))
    for t in range(6):
        np.add.at(M, (np.arange(D), fold[:, t]), wts[:, t])
    KL = np.einsum('hk,wl->klhw', M, M).reshape(g * g, HW)
    return (jnp.asarray(M, dtype=jnp.float32),
            jnp.asarray(KL, dtype=jnp.float32))


def _fused_kernel(m_ref, tiles_ref, slices_ref, kl_ref, out_ref,
                  cdf_s, u_s, mm_s):
    g = pl.program_id(0)

    @pl.when(g == 0)
    def _init():
        cdf_s[...] = jnp.zeros((NEXT, N_TILES), jnp.float32)
        mm_s[0] = jnp.float32(jnp.inf)
        mm_s[1] = jnp.float32(-jnp.inf)

    @pl.when(g < PH1)
    def _hist():
        for r in range(TB):
            t_idx = g * TB + r
            v = tiles_ref[r]
            bins_col = jax.lax.broadcasted_iota(
                jnp.int32, (N_BINS, 1), 0).astype(jnp.float32) * (
                    1.0 / (N_BINS - 1))
            dv = v - bins_col
            wts = jnp.exp((dv * dv) * (-0.5 / (BANDWIDTH * BANDWIDTH)))
            pdf = jnp.sum(wts, axis=1, keepdims=True) * (1.0 / VOXELS)
            pdf = pdf / (jnp.sum(pdf) + 1e-10)
            histos = jnp.minimum(pdf * VOXELS, LIMIT)
            clipped = VOXELS - jnp.sum(histos)
            redist = jnp.floor(clipped * (1.0 / N_BINS))
            residual = clipped - redist * N_BINS
            iota = jax.lax.broadcasted_iota(
                jnp.int32, (N_BINS, 1), 0).astype(jnp.float32)
            histos = histos + redist + (iota < residual).astype(jnp.float32)
            rr = jax.lax.broadcasted_iota(jnp.int32, (N_BINS, N_BINS), 0)
            cc = jax.lax.broadcasted_iota(jnp.int32, (N_BINS, N_BINS), 1)
            ltri = (rr >= cc).astype(jnp.float32)
            cdf = jax.lax.dot_general(
                ltri, histos, (((1,), (0,)), ((), ())),
                preferred_element_type=jnp.float32)
            cdf = cdf * ((N_BINS - 1.0) / VOXELS)
            ext = jnp.concatenate(
                [cdf[3:4], cdf[2:3], cdf[1:2], cdf[0:1], cdf,
                 cdf[63:64], cdf[62:63], cdf[61:62], cdf[60:61]], axis=0)
            lane = jax.lax.broadcasted_iota(jnp.int32, (NEXT, N_TILES), 1)
            cdf_s[...] += ext * (lane == t_idx).astype(jnp.float32)

    @pl.when(jnp.logical_and(g >= PH1, g < PH2))
    def _apply():
        kl = kl_ref[...]
        j_icol = jax.lax.broadcasted_iota(jnp.int32, (NEXT, 1), 0)
        for r in range(SB):
            d = (g - PH1) * SB + r
            p0 = slices_ref[r] * (N_BINS - 1.0)
            base = jnp.floor(p0)
            f = p0 - base
            bi = base.astype(jnp.int32)
            onehot = (j_icol == bi + 2).astype(jnp.float32)
            c_d = (m_ref[d, 0] * cdf_s[:, 0:16]
                   + m_ref[d, 1] * cdf_s[:, 16:32]
                   + m_ref[d, 2] * cdf_s[:, 32:48]
                   + m_ref[d, 3] * cdf_s[:, 48:64])
            b = None
            for t in range(6):
                if t == 0:
                    c_t = c_d
                else:
                    c_t = jnp.concatenate(
                        [c_d[t:], jnp.zeros((t, 16), jnp.float32)], axis=0)
                s_t = jax.lax.dot_general(
                    c_t, onehot, (((0,), (0,)), ((), ())),
                    preferred_element_type=jnp.float32)
                w_t = _WC[t, 0] + f * (
                    _WC[t, 1] + f * (_WC[t, 2] + f * (
                        _WC[t, 3] + f * (_WC[t, 4] + f * _WC[t, 5]))))
                b = w_t * s_t if b is None else b + w_t * s_t
            row = jnp.sum(b * kl, axis=0, keepdims=True)
            u_s[pl.ds(d, 1), :] = row
            mm_s[0] = jnp.minimum(mm_s[0], jnp.min(row))
            mm_s[1] = jnp.maximum(mm_s[1], jnp.max(row))

    @pl.when(g >= PH2)
    def _norm():
        d0 = (g - PH2) * NB
        gmin = mm_s[0]
        gmax = mm_s[1]
        out_ref[:, 0, :] = (u_s[pl.ds(d0, NB), :] - gmin) / (
            gmax - gmin + 1e-10)


def kernel(x):
    gd, gh, gw = GRID
    td, th, tw = D // gd, H // gh, W // gw
    xf = x.reshape(D, H, W)
    tiles = xf.reshape(gd, td, gh, th, gw, tw).transpose(
        0, 2, 4, 1, 3, 5).reshape(N_TILES, 1, VOXELS)
    slices = xf.reshape(D, 1, HW)
    M, KL = _spatial_consts()

    out = pl.pallas_call(
        _fused_kernel,
        grid=(PH3,),
        in_specs=[
            pl.BlockSpec(memory_space=pltpu.SMEM),
            pl.BlockSpec((TB, 1, VOXELS),
                         lambda g: (jnp.minimum(g, PH1 - 1), 0, 0)),
            pl.BlockSpec((SB, 1, HW),
                         lambda g: (jnp.clip(g - PH1, 0, D // SB - 1), 0, 0)),
            pl.BlockSpec((gh * gw, HW), lambda g: (0, 0)),
        ],
        out_specs=pl.BlockSpec((NB, 1, HW),
                               lambda g: (jnp.clip(g - PH2, 0,
                                                   D // NB - 1), 0, 0)),
        out_shape=jax.ShapeDtypeStruct((D, 1, HW), jnp.float32),
        scratch_shapes=[
            pltpu.VMEM((NEXT, N_TILES), jnp.float32),
            pltpu.VMEM((D, HW), jnp.float32),
            pltpu.SMEM((2,), jnp.float32),
        ],
    )(M, tiles, slices, KL)

    return out.reshape(1, 1, D, H, W)


# speedup vs baseline: 53366.0682x; 1.3276x over previous
import functools
import math

import jax
import jax.numpy as jnp
import numpy as np
from jax.experimental import pallas as pl
from jax.experimental.pallas import tpu as pltpu

N_BINS = 64
GRID = (4, 4, 4)
BANDWIDTH = 1e-3
CLIP_LIMIT = 4.0
D = H = W = 64
VOXELS = (D // GRID[0]) * (H // GRID[1]) * (W // GRID[2])
LIMIT = max(CLIP_LIMIT * VOXELS // N_BINS, 1.0)
NEXT = N_BINS + 8
N_TILES = GRID[0] * GRID[1] * GRID[2]
HW = H * W
TB = 4
SB = 4
NB = 16
PH1 = N_TILES // TB
PH2 = PH1 + D // SB
PH3 = PH2 + D // NB


def _b5_np(t):
    x = np.abs(t)
    p3 = np.clip(3.0 - x, 0.0, None) ** 5
    p2 = np.clip(2.0 - x, 0.0, None) ** 5
    p1 = np.clip(1.0 - x, 0.0, None) ** 5
    return (p3 - 6.0 * p2 + 15.0 * p1) / 120.0


def _fold_np(i, n):
    j = np.mod(i, 2 * n)
    return np.where(j >= n, 2 * n - 1 - j, j)


def _tap_coeffs():
    def pc(a, sign):
        return np.array([math.comb(5, k) * a ** (5 - k) * sign ** k
                         for k in range(6)], dtype=np.float64)
    Wc = np.zeros((6, 6))
    Wc[0] = pc(1, -1) / 120.0
    Wc[1] = (pc(2, -1) - 6 * pc(1, -1)) / 120.0
    Wc[2] = (pc(3, -1) - 6 * pc(2, -1) + 15 * pc(1, -1)) / 120.0
    Wc[3] = (pc(2, 1) - 6 * pc(1, 1) + 15 * pc(0, 1)) / 120.0
    Wc[4] = (pc(1, 1) - 6 * pc(0, 1)) / 120.0
    Wc[5] = pc(0, 1) / 120.0
    return Wc.astype(np.float32)


_WC = _tap_coeffs()


@functools.lru_cache(maxsize=None)
def _spatial_consts():
    g = GRID[0]
    c = np.linspace(-0.5 - 0.25 / g, g - 1 + 0.5 + 0.25 / g, D)
    base = np.floor(c)
    offs = np.arange(-2, 4)
    idx = base[:, None] + offs[None, :]
    wts = _b5_np(c[:, None] - idx)
    fold = _fold_np(idx.astype(np.int64), g)
    M = np.zeros((D, g))
    for t in range(6):
        np.add.at(M, (np.arange(D), fold[:, t]), wts[:, t])
    KL = np.einsum('hk,wl->klhw', M, M).reshape(g * g, HW)
    return (jnp.asarray(M, dtype=jnp.float32),
            jnp.asarray(KL, dtype=jnp.float32))


def _fused_kernel(m_ref, tiles_ref, slices_ref, kl_ref, out_ref,
                  cdf_s, u_s, mm_s):
    g = pl.program_id(0)

    @pl.when(g == 0)
    def _init():
        cdf_s[...] = jnp.zeros((NEXT, N_TILES), jnp.float32)
        mm_s[0] = jnp.float32(jnp.inf)
        mm_s[1] = jnp.float32(-jnp.inf)

    @pl.when(g < PH1)
    def _hist():
        for r in range(TB):
            t_idx = g * TB + r
            v = tiles_ref[r]
            bins_col = jax.lax.broadcasted_iota(
                jnp.int32, (N_BINS, 1), 0).astype(jnp.float32) * (
                    1.0 / (N_BINS - 1))
            dv = v - bins_col
            wts = jnp.exp((dv * dv) * (-0.5 / (BANDWIDTH * BANDWIDTH)))
            pdf = jnp.sum(wts, axis=1, keepdims=True) * (1.0 / VOXELS)
            pdf = pdf / (jnp.sum(pdf) + 1e-10)
            histos = jnp.minimum(pdf * VOXELS, LIMIT)
            clipped = VOXELS - jnp.sum(histos)
            redist = jnp.floor(clipped * (1.0 / N_BINS))
            residual = clipped - redist * N_BINS
            iota = jax.lax.broadcasted_iota(
                jnp.int32, (N_BINS, 1), 0).astype(jnp.float32)
            histos = histos + redist + (iota < residual).astype(jnp.float32)
            rr = jax.lax.broadcasted_iota(jnp.int32, (N_BINS, N_BINS), 0)
            cc = jax.lax.broadcasted_iota(jnp.int32, (N_BINS, N_BINS), 1)
            ltri = (rr >= cc).astype(jnp.float32)
            cdf = jax.lax.dot_general(
                ltri, histos, (((1,), (0,)), ((), ())),
                preferred_element_type=jnp.float32)
            cdf = cdf * ((N_BINS - 1.0) / VOXELS)
            ext = jnp.concatenate(
                [cdf[3:4], cdf[2:3], cdf[1:2], cdf[0:1], cdf,
                 cdf[63:64], cdf[62:63], cdf[61:62], cdf[60:61]], axis=0)
            lane = jax.lax.broadcasted_iota(jnp.int32, (NEXT, N_TILES), 1)
            cdf_s[...] += ext * (lane == t_idx).astype(jnp.float32)

    @pl.when(jnp.logical_and(g >= PH1, g < PH2))
    def _apply():
        kl = kl_ref[...]
        j_icol = jax.lax.broadcasted_iota(jnp.int32, (NEXT, 1), 0)
        for r in range(SB):
            d = (g - PH1) * SB + r
            p0 = slices_ref[r] * (N_BINS - 1.0)
            base = jnp.floor(p0)
            f = p0 - base
            bi = base.astype(jnp.int32)
            onehot = (j_icol == bi + 2).astype(jnp.float32)
            c_d = (m_ref[d, 0] * cdf_s[:, 0:16]
                   + m_ref[d, 1] * cdf_s[:, 16:32]
                   + m_ref[d, 2] * cdf_s[:, 32:48]
                   + m_ref[d, 3] * cdf_s[:, 48:64])
            c6 = jnp.concatenate(
                [c_d] + [jnp.concatenate(
                    [c_d[t:], jnp.zeros((t, 16), jnp.float32)], axis=0)
                    for t in range(1, 6)], axis=1)
            s6 = jax.lax.dot_general(
                c6, onehot, (((0,), (0,)), ((), ())),
                preferred_element_type=jnp.float32)
            b = None
            for t in range(6):
                w_t = _WC[t, 0] + f * (
                    _WC[t, 1] + f * (_WC[t, 2] + f * (
                        _WC[t, 3] + f * (_WC[t, 4] + f * _WC[t, 5]))))
                s_t = s6[16 * t:16 * (t + 1)]
                b = w_t * s_t if b is None else b + w_t * s_t
            row = jnp.sum(b * kl, axis=0, keepdims=True)
            u_s[pl.ds(d, 1), :] = row
            mm_s[0] = jnp.minimum(mm_s[0], jnp.min(row))
            mm_s[1] = jnp.maximum(mm_s[1], jnp.max(row))

    @pl.when(g >= PH2)
    def _norm():
        d0 = (g - PH2) * NB
        gmin = mm_s[0]
        gmax = mm_s[1]
        out_ref[:, 0, :] = (u_s[pl.ds(d0, NB), :] - gmin) / (
            gmax - gmin + 1e-10)


def kernel(x):
    gd, gh, gw = GRID
    td, th, tw = D // gd, H // gh, W // gw
    xf = x.reshape(D, H, W)
    tiles = xf.reshape(gd, td, gh, th, gw, tw).transpose(
        0, 2, 4, 1, 3, 5).reshape(N_TILES, 1, VOXELS)
    slices = xf.reshape(D, 1, HW)
    M, KL = _spatial_consts()

    out = pl.pallas_call(
        _fused_kernel,
        grid=(PH3,),
        in_specs=[
            pl.BlockSpec(memory_space=pltpu.SMEM),
            pl.BlockSpec((TB, 1, VOXELS),
                         lambda g: (jnp.minimum(g, PH1 - 1), 0, 0)),
            pl.BlockSpec((SB, 1, HW),
                         lambda g: (jnp.clip(g - PH1, 0, D // SB - 1), 0, 0)),
            pl.BlockSpec((gh * gw, HW), lambda g: (0, 0)),
        ],
        out_specs=pl.BlockSpec((NB, 1, HW),
                               lambda g: (jnp.clip(g - PH2, 0,
                                                   D // NB - 1), 0, 0)),
        out_shape=jax.ShapeDtypeStruct((D, 1, HW), jnp.float32),
        scratch_shapes=[
            pltpu.VMEM((NEXT, N_TILES), jnp.float32),
            pltpu.VMEM((D, HW), jnp.float32),
            pltpu.SMEM((2,), jnp.float32),
        ],
    )(M, tiles, slices, KL)

    return out.reshape(1, 1, D, H, W)


# speedup vs baseline: 59242.2590x; 1.1101x over previous
import functools
import math

import jax
import jax.numpy as jnp
import numpy as np
from jax.experimental import pallas as pl
from jax.experimental.pallas import tpu as pltpu

N_BINS = 64
GRID = (4, 4, 4)
BANDWIDTH = 1e-3
CLIP_LIMIT = 4.0
D = H = W = 64
VOXELS = (D // GRID[0]) * (H // GRID[1]) * (W // GRID[2])
LIMIT = max(CLIP_LIMIT * VOXELS // N_BINS, 1.0)
NEXT = N_BINS + 8
N_TILES = GRID[0] * GRID[1] * GRID[2]
HW = H * W
TB = 8
SB = 8
NB = 32
PH1 = N_TILES // TB
PH2 = PH1 + D // SB
PH3 = PH2 + D // NB


def _b5_np(t):
    x = np.abs(t)
    p3 = np.clip(3.0 - x, 0.0, None) ** 5
    p2 = np.clip(2.0 - x, 0.0, None) ** 5
    p1 = np.clip(1.0 - x, 0.0, None) ** 5
    return (p3 - 6.0 * p2 + 15.0 * p1) / 120.0


def _fold_np(i, n):
    j = np.mod(i, 2 * n)
    return np.where(j >= n, 2 * n - 1 - j, j)


def _tap_coeffs():
    def pc(a, sign):
        return np.array([math.comb(5, k) * a ** (5 - k) * sign ** k
                         for k in range(6)], dtype=np.float64)
    Wc = np.zeros((6, 6))
    Wc[0] = pc(1, -1) / 120.0
    Wc[1] = (pc(2, -1) - 6 * pc(1, -1)) / 120.0
    Wc[2] = (pc(3, -1) - 6 * pc(2, -1) + 15 * pc(1, -1)) / 120.0
    Wc[3] = (pc(2, 1) - 6 * pc(1, 1) + 15 * pc(0, 1)) / 120.0
    Wc[4] = (pc(1, 1) - 6 * pc(0, 1)) / 120.0
    Wc[5] = pc(0, 1) / 120.0
    return Wc.astype(np.float32)


_WC = _tap_coeffs()


@functools.lru_cache(maxsize=None)
def _spatial_consts():
    g = GRID[0]
    c = np.linspace(-0.5 - 0.25 / g, g - 1 + 0.5 + 0.25 / g, D)
    base = np.floor(c)
    offs = np.arange(-2, 4)
    idx = base[:, None] + offs[None, :]
    wts = _b5_np(c[:, None] - idx)
    fold = _fold_np(idx.astype(np.int64), g)
    M = np.zeros((D, g))
    for t in range(6):
        np.add.at(M, (np.arange(D), fold[:, t]), wts[:, t])
    KL = np.einsum('hk,wl->klhw', M, M).reshape(g * g, HW)
    return (jnp.asarray(M, dtype=jnp.float32),
            jnp.asarray(KL, dtype=jnp.float32))


def _fused_kernel(m_ref, tiles_ref, slices_ref, kl_ref, out_ref,
                  cdf_s, u_s, mm_s):
    g = pl.program_id(0)

    @pl.when(g == 0)
    def _init():
        cdf_s[...] = jnp.zeros((NEXT, N_TILES), jnp.float32)
        mm_s[0] = jnp.float32(jnp.inf)
        mm_s[1] = jnp.float32(-jnp.inf)

    @pl.when(g < PH1)
    def _hist():
        for r in range(TB):
            t_idx = g * TB + r
            v = tiles_ref[r]
            bins_col = jax.lax.broadcasted_iota(
                jnp.int32, (N_BINS, 1), 0).astype(jnp.float32) * (
                    1.0 / (N_BINS - 1))
            dv = v - bins_col
            wts = jnp.exp((dv * dv) * (-0.5 / (BANDWIDTH * BANDWIDTH)))
            pdf = jnp.sum(wts, axis=1, keepdims=True) * (1.0 / VOXELS)
            pdf = pdf / (jnp.sum(pdf) + 1e-10)
            histos = jnp.minimum(pdf * VOXELS, LIMIT)
            clipped = VOXELS - jnp.sum(histos)
            redist = jnp.floor(clipped * (1.0 / N_BINS))
            residual = clipped - redist * N_BINS
            iota = jax.lax.broadcasted_iota(
                jnp.int32, (N_BINS, 1), 0).astype(jnp.float32)
            histos = histos + redist + (iota < residual).astype(jnp.float32)
            rr = jax.lax.broadcasted_iota(jnp.int32, (N_BINS, N_BINS), 0)
            cc = jax.lax.broadcasted_iota(jnp.int32, (N_BINS, N_BINS), 1)
            ltri = (rr >= cc).astype(jnp.float32)
            cdf = jax.lax.dot_general(
                ltri, histos, (((1,), (0,)), ((), ())),
                preferred_element_type=jnp.float32)
            cdf = cdf * ((N_BINS - 1.0) / VOXELS)
            ext = jnp.concatenate(
                [cdf[3:4], cdf[2:3], cdf[1:2], cdf[0:1], cdf,
                 cdf[63:64], cdf[62:63], cdf[61:62], cdf[60:61]], axis=0)
            lane = jax.lax.broadcasted_iota(jnp.int32, (NEXT, N_TILES), 1)
            cdf_s[...] += ext * (lane == t_idx).astype(jnp.float32)

    @pl.when(jnp.logical_and(g >= PH1, g < PH2))
    def _apply():
        kl = kl_ref[...]
        j_icol = jax.lax.broadcasted_iota(jnp.int32, (NEXT, 1), 0)
        for r in range(SB):
            d = (g - PH1) * SB + r
            p0 = slices_ref[r] * (N_BINS - 1.0)
            base = jnp.floor(p0)
            f = p0 - base
            bi = base.astype(jnp.int32)
            onehot = (j_icol == bi + 2).astype(jnp.float32)
            c_d = (m_ref[d, 0] * cdf_s[:, 0:16]
                   + m_ref[d, 1] * cdf_s[:, 16:32]
                   + m_ref[d, 2] * cdf_s[:, 32:48]
                   + m_ref[d, 3] * cdf_s[:, 48:64])
            c6 = jnp.concatenate(
                [c_d] + [jnp.concatenate(
                    [c_d[t:], jnp.zeros((t, 16), jnp.float32)], axis=0)
                    for t in range(1, 6)], axis=1)
            s6 = jax.lax.dot_general(
                c6, onehot, (((0,), (0,)), ((), ())),
                preferred_element_type=jnp.float32)
            b = None
            for t in range(6):
                w_t = _WC[t, 0] + f * (
                    _WC[t, 1] + f * (_WC[t, 2] + f * (
                        _WC[t, 3] + f * (_WC[t, 4] + f * _WC[t, 5]))))
                s_t = s6[16 * t:16 * (t + 1)]
                b = w_t * s_t if b is None else b + w_t * s_t
            row = jnp.sum(b * kl, axis=0, keepdims=True)
            u_s[pl.ds(d, 1), :] = row
            mm_s[0] = jnp.minimum(mm_s[0], jnp.min(row))
            mm_s[1] = jnp.maximum(mm_s[1], jnp.max(row))

    @pl.when(g >= PH2)
    def _norm():
        d0 = (g - PH2) * NB
        gmin = mm_s[0]
        gmax = mm_s[1]
        out_ref[:, 0, :] = (u_s[pl.ds(d0, NB), :] - gmin) / (
            gmax - gmin + 1e-10)


def kernel(x):
    gd, gh, gw = GRID
    td, th, tw = D // gd, H // gh, W // gw
    xf = x.reshape(D, H, W)
    tiles = xf.reshape(gd, td, gh, th, gw, tw).transpose(
        0, 2, 4, 1, 3, 5).reshape(N_TILES, 1, VOXELS)
    slices = xf.reshape(D, 1, HW)
    M, KL = _spatial_consts()

    out = pl.pallas_call(
        _fused_kernel,
        grid=(PH3,),
        in_specs=[
            pl.BlockSpec(memory_space=pltpu.SMEM),
            pl.BlockSpec((TB, 1, VOXELS),
                         lambda g: (jnp.minimum(g, PH1 - 1), 0, 0)),
            pl.BlockSpec((SB, 1, HW),
                         lambda g: (jnp.clip(g - PH1, 0, D // SB - 1), 0, 0)),
            pl.BlockSpec((gh * gw, HW), lambda g: (0, 0)),
        ],
        out_specs=pl.BlockSpec((NB, 1, HW),
                               lambda g: (jnp.clip(g - PH2, 0,
                                                   D // NB - 1), 0, 0)),
        out_shape=jax.ShapeDtypeStruct((D, 1, HW), jnp.float32),
        scratch_shapes=[
            pltpu.VMEM((NEXT, N_TILES), jnp.float32),
            pltpu.VMEM((D, HW), jnp.float32),
            pltpu.SMEM((2,), jnp.float32),
        ],
    )(M, tiles, slices, KL)

    return out.reshape(1, 1, D, H, W)


# speedup vs baseline: 62210.8861x; 1.0501x over previous
import functools
import math

import jax
import jax.numpy as jnp
import numpy as np
from jax.experimental import pallas as pl
from jax.experimental.pallas import tpu as pltpu

N_BINS = 64
GRID = (4, 4, 4)
BANDWIDTH = 1e-3
CLIP_LIMIT = 4.0
D = H = W = 64
VOXELS = (D // GRID[0]) * (H // GRID[1]) * (W // GRID[2])
LIMIT = max(CLIP_LIMIT * VOXELS // N_BINS, 1.0)
NEXT = N_BINS + 8
N_TILES = GRID[0] * GRID[1] * GRID[2]
HW = H * W
TB = 16
SB = 16
NB = 64
PH1 = N_TILES // TB
PH2 = PH1 + D // SB
PH3 = PH2 + D // NB


def _b5_np(t):
    x = np.abs(t)
    p3 = np.clip(3.0 - x, 0.0, None) ** 5
    p2 = np.clip(2.0 - x, 0.0, None) ** 5
    p1 = np.clip(1.0 - x, 0.0, None) ** 5
    return (p3 - 6.0 * p2 + 15.0 * p1) / 120.0


def _fold_np(i, n):
    j = np.mod(i, 2 * n)
    return np.where(j >= n, 2 * n - 1 - j, j)


def _tap_coeffs():
    def pc(a, sign):
        return np.array([math.comb(5, k) * a ** (5 - k) * sign ** k
                         for k in range(6)], dtype=np.float64)
    Wc = np.zeros((6, 6))
    Wc[0] = pc(1, -1) / 120.0
    Wc[1] = (pc(2, -1) - 6 * pc(1, -1)) / 120.0
    Wc[2] = (pc(3, -1) - 6 * pc(2, -1) + 15 * pc(1, -1)) / 120.0
    Wc[3] = (pc(2, 1) - 6 * pc(1, 1) + 15 * pc(0, 1)) / 120.0
    Wc[4] = (pc(1, 1) - 6 * pc(0, 1)) / 120.0
    Wc[5] = pc(0, 1) / 120.0
    return Wc.astype(np.float32)


_WC = _tap_coeffs()


@functools.lru_cache(maxsize=None)
def _spatial_consts():
    g = GRID[0]
    c = np.linspace(-0.5 - 0.25 / g, g - 1 + 0.5 + 0.25 / g, D)
    base = np.floor(c)
    offs = np.arange(-2, 4)
    idx = base[:, None] + offs[None, :]
    wts = _b5_np(c[:, None] - idx)
    fold = _fold_np(idx.astype(np.int64), g)
    M = np.zeros((D, g))
    for t in range(6):
        np.add.at(M, (np.arange(D), fold[:, t]), wts[:, t])
    KL = np.einsum('hk,wl->klhw', M, M).reshape(g * g, HW)
    return (jnp.asarray(M, dtype=jnp.float32),
            jnp.asarray(KL, dtype=jnp.float32))


def _fused_kernel(m_ref, tiles_ref, slices_ref, kl_ref, out_ref,
                  cdf_s, u_s, mm_s):
    g = pl.program_id(0)

    @pl.when(g == 0)
    def _init():
        cdf_s[...] = jnp.zeros((NEXT, N_TILES), jnp.float32)
        mm_s[0] = jnp.float32(jnp.inf)
        mm_s[1] = jnp.float32(-jnp.inf)

    @pl.when(g < PH1)
    def _hist():
        for r in range(TB):
            t_idx = g * TB + r
            v = tiles_ref[r]
            bins_col = jax.lax.broadcasted_iota(
                jnp.int32, (N_BINS, 1), 0).astype(jnp.float32) * (
                    1.0 / (N_BINS - 1))
            dv = v - bins_col
            wts = jnp.exp((dv * dv) * (-0.5 / (BANDWIDTH * BANDWIDTH)))
            pdf = jnp.sum(wts, axis=1, keepdims=True) * (1.0 / VOXELS)
            pdf = pdf / (jnp.sum(pdf) + 1e-10)
            histos = jnp.minimum(pdf * VOXELS, LIMIT)
            clipped = VOXELS - jnp.sum(histos)
            redist = jnp.floor(clipped * (1.0 / N_BINS))
            residual = clipped - redist * N_BINS
            iota = jax.lax.broadcasted_iota(
                jnp.int32, (N_BINS, 1), 0).astype(jnp.float32)
            histos = histos + redist + (iota < residual).astype(jnp.float32)
            rr = jax.lax.broadcasted_iota(jnp.int32, (N_BINS, N_BINS), 0)
            cc = jax.lax.broadcasted_iota(jnp.int32, (N_BINS, N_BINS), 1)
            ltri = (rr >= cc).astype(jnp.float32)
            cdf = jax.lax.dot_general(
                ltri, histos, (((1,), (0,)), ((), ())),
                preferred_element_type=jnp.float32)
            cdf = cdf * ((N_BINS - 1.0) / VOXELS)
            ext = jnp.concatenate(
                [cdf[3:4], cdf[2:3], cdf[1:2], cdf[0:1], cdf,
                 cdf[63:64], cdf[62:63], cdf[61:62], cdf[60:61]], axis=0)
            lane = jax.lax.broadcasted_iota(jnp.int32, (NEXT, N_TILES), 1)
            cdf_s[...] += ext * (lane == t_idx).astype(jnp.float32)

    @pl.when(jnp.logical_and(g >= PH1, g < PH2))
    def _apply():
        kl = kl_ref[...]
        j_icol = jax.lax.broadcasted_iota(jnp.int32, (NEXT, 1), 0)
        for r in range(SB):
            d = (g - PH1) * SB + r
            p0 = slices_ref[r] * (N_BINS - 1.0)
            base = jnp.floor(p0)
            f = p0 - base
            bi = base.astype(jnp.int32)
            onehot = (j_icol == bi + 2).astype(jnp.float32)
            c_d = (m_ref[d, 0] * cdf_s[:, 0:16]
                   + m_ref[d, 1] * cdf_s[:, 16:32]
                   + m_ref[d, 2] * cdf_s[:, 32:48]
                   + m_ref[d, 3] * cdf_s[:, 48:64])
            c6 = jnp.concatenate(
                [c_d] + [jnp.concatenate(
                    [c_d[t:], jnp.zeros((t, 16), jnp.float32)], axis=0)
                    for t in range(1, 6)], axis=1)
            s6 = jax.lax.dot_general(
                c6, onehot, (((0,), (0,)), ((), ())),
                preferred_element_type=jnp.float32)
            b = None
            for t in range(6):
                w_t = _WC[t, 0] + f * (
                    _WC[t, 1] + f * (_WC[t, 2] + f * (
                        _WC[t, 3] + f * (_WC[t, 4] + f * _WC[t, 5]))))
                s_t = s6[16 * t:16 * (t + 1)]
                b = w_t * s_t if b is None else b + w_t * s_t
            row = jnp.sum(b * kl, axis=0, keepdims=True)
            u_s[pl.ds(d, 1), :] = row
            mm_s[0] = jnp.minimum(mm_s[0], jnp.min(row))
            mm_s[1] = jnp.maximum(mm_s[1], jnp.max(row))

    @pl.when(g >= PH2)
    def _norm():
        d0 = (g - PH2) * NB
        gmin = mm_s[0]
        gmax = mm_s[1]
        out_ref[:, 0, :] = (u_s[pl.ds(d0, NB), :] - gmin) / (
            gmax - gmin + 1e-10)


def kernel(x):
    gd, gh, gw = GRID
    td, th, tw = D // gd, H // gh, W // gw
    xf = x.reshape(D, H, W)
    tiles = xf.reshape(gd, td, gh, th, gw, tw).transpose(
        0, 2, 4, 1, 3, 5).reshape(N_TILES, 1, VOXELS)
    slices = xf.reshape(D, 1, HW)
    M, KL = _spatial_consts()

    out = pl.pallas_call(
        _fused_kernel,
        grid=(PH3,),
        in_specs=[
            pl.BlockSpec(memory_space=pltpu.SMEM),
            pl.BlockSpec((TB, 1, VOXELS),
                         lambda g: (jnp.minimum(g, PH1 - 1), 0, 0)),
            pl.BlockSpec((SB, 1, HW),
                         lambda g: (jnp.clip(g - PH1, 0, D // SB - 1), 0, 0)),
            pl.BlockSpec((gh * gw, HW), lambda g: (0, 0)),
        ],
        out_specs=pl.BlockSpec((NB, 1, HW),
                               lambda g: (jnp.clip(g - PH2, 0,
                                                   D // NB - 1), 0, 0)),
        out_shape=jax.ShapeDtypeStruct((D, 1, HW), jnp.float32),
        scratch_shapes=[
            pltpu.VMEM((NEXT, N_TILES), jnp.float32),
            pltpu.VMEM((D, HW), jnp.float32),
            pltpu.SMEM((2,), jnp.float32),
        ],
    )(M, tiles, slices, KL)

    return out.reshape(1, 1, D, H, W)


# speedup vs baseline: 87198.2917x; 1.4017x over previous
import functools
import math

import jax
import jax.numpy as jnp
import numpy as np
from jax.experimental import pallas as pl
from jax.experimental.pallas import tpu as pltpu

N_BINS = 64
GRID = (4, 4, 4)
BANDWIDTH = 1e-3
CLIP_LIMIT = 4.0
D = H = W = 64
VOXELS = (D // GRID[0]) * (H // GRID[1]) * (W // GRID[2])
LIMIT = max(CLIP_LIMIT * VOXELS // N_BINS, 1.0)
NEXT = N_BINS + 8
N_TILES = GRID[0] * GRID[1] * GRID[2]
HW = H * W
TD = D // GRID[0]
SB = 16
NB = 64
PH1 = GRID[0]
PH2 = PH1 + D // SB
PH3 = PH2 + D // NB


def _b5_np(t):
    x = np.abs(t)
    p3 = np.clip(3.0 - x, 0.0, None) ** 5
    p2 = np.clip(2.0 - x, 0.0, None) ** 5
    p1 = np.clip(1.0 - x, 0.0, None) ** 5
    return (p3 - 6.0 * p2 + 15.0 * p1) / 120.0


def _fold_np(i, n):
    j = np.mod(i, 2 * n)
    return np.where(j >= n, 2 * n - 1 - j, j)


def _tap_coeffs():
    def pc(a, sign):
        return np.array([math.comb(5, k) * a ** (5 - k) * sign ** k
                         for k in range(6)], dtype=np.float64)
    Wc = np.zeros((6, 6))
    Wc[0] = pc(1, -1) / 120.0
    Wc[1] = (pc(2, -1) - 6 * pc(1, -1)) / 120.0
    Wc[2] = (pc(3, -1) - 6 * pc(2, -1) + 15 * pc(1, -1)) / 120.0
    Wc[3] = (pc(2, 1) - 6 * pc(1, 1) + 15 * pc(0, 1)) / 120.0
    Wc[4] = (pc(1, 1) - 6 * pc(0, 1)) / 120.0
    Wc[5] = pc(0, 1) / 120.0
    return Wc.astype(np.float32)


_WC = _tap_coeffs()


@functools.lru_cache(maxsize=None)
def _spatial_consts():
    g = GRID[0]
    c = np.linspace(-0.5 - 0.25 / g, g - 1 + 0.5 + 0.25 / g, D)
    base = np.floor(c)
    offs = np.arange(-2, 4)
    idx = base[:, None] + offs[None, :]
    wts = _b5_np(c[:, None] - idx)
    fold = _fold_np(idx.astype(np.int64), g)
    M = np.zeros((D, g))
    for t in range(6):
        np.add.at(M, (np.arange(D), fold[:, t]), wts[:, t])
    KL = np.einsum('hk,wl->klhw', M, M).reshape(g * g, HW)
    hh, ww = np.meshgrid(np.arange(H), np.arange(W), indexing='ij')
    kl_of = (hh // (H // g)) * g + (ww // (W // g))
    S = (kl_of.reshape(HW, 1) == np.arange(g * g)[None, :]).astype(
        np.float32)
    return (jnp.asarray(M, dtype=jnp.float32),
            jnp.asarray(KL, dtype=jnp.float32),
            jnp.asarray(S))


def _fused_kernel(m_ref, slices_ref, kl_ref, s_ref, out_ref,
                  cdf_s, u_s, mm_s):
    g = pl.program_id(0)

    @pl.when(g == 0)
    def _init():
        cdf_s[...] = jnp.zeros((NEXT, N_TILES), jnp.float32)
        mm_s[0] = jnp.float32(jnp.inf)
        mm_s[1] = jnp.float32(-jnp.inf)

    @pl.when(g < PH1)
    def _hist():
        smat = s_ref[...]
        bins_col = jax.lax.broadcasted_iota(
            jnp.int32, (N_BINS, 1), 0).astype(jnp.float32) * (
                1.0 / (N_BINS - 1))
        psum = jnp.zeros((N_BINS, GRID[1] * GRID[2]), jnp.float32)
        for r in range(TD):
            v = slices_ref[r]
            dv = v - bins_col
            wts = jnp.exp((dv * dv) * (-0.5 / (BANDWIDTH * BANDWIDTH)))
            psum = psum + jax.lax.dot_general(
                wts, smat, (((1,), (0,)), ((), ())),
                preferred_element_type=jnp.float32)
        pdf = psum * (1.0 / VOXELS)
        pdf = pdf / (jnp.sum(pdf, axis=0, keepdims=True) + 1e-10)
        histos = jnp.minimum(pdf * VOXELS, LIMIT)
        clipped = VOXELS - jnp.sum(histos, axis=0, keepdims=True)
        redist = jnp.floor(clipped * (1.0 / N_BINS))
        residual = clipped - redist * N_BINS
        iota = jax.lax.broadcasted_iota(
            jnp.int32, (N_BINS, 1), 0).astype(jnp.float32)
        histos = histos + redist + (iota < residual).astype(jnp.float32)
        rr = jax.lax.broadcasted_iota(jnp.int32, (N_BINS, N_BINS), 0)
        cc = jax.lax.broadcasted_iota(jnp.int32, (N_BINS, N_BINS), 1)
        ltri = (rr >= cc).astype(jnp.float32)
        cdf = jax.lax.dot_general(
            ltri, histos, (((1,), (0,)), ((), ())),
            preferred_element_type=jnp.float32)
        cdf = cdf * ((N_BINS - 1.0) / VOXELS)
        ext = jnp.concatenate(
            [cdf[3:4], cdf[2:3], cdf[1:2], cdf[0:1], cdf,
             cdf[63:64], cdf[62:63], cdf[61:62], cdf[60:61]], axis=0)
        ext4 = jnp.concatenate([ext, ext, ext, ext], axis=1)
        lane = jax.lax.broadcasted_iota(jnp.int32, (NEXT, N_TILES), 1)
        in_blk = jnp.logical_and(lane >= g * 16, lane < g * 16 + 16)
        cdf_s[...] += ext4 * in_blk.astype(jnp.float32)

    @pl.when(jnp.logical_and(g >= PH1, g < PH2))
    def _apply():
        kl = kl_ref[...]
        j_icol = jax.lax.broadcasted_iota(jnp.int32, (NEXT, 1), 0)
        for r in range(SB):
            d = (g - PH1) * SB + r
            p0 = slices_ref[r] * (N_BINS - 1.0)
            base = jnp.floor(p0)
            f = p0 - base
            bi = base.astype(jnp.int32)
            onehot = (j_icol == bi + 2).astype(jnp.float32)
            c_d = (m_ref[d, 0] * cdf_s[:, 0:16]
                   + m_ref[d, 1] * cdf_s[:, 16:32]
                   + m_ref[d, 2] * cdf_s[:, 32:48]
                   + m_ref[d, 3] * cdf_s[:, 48:64])
            c6 = jnp.concatenate(
                [c_d] + [jnp.concatenate(
                    [c_d[t:], jnp.zeros((t, 16), jnp.float32)], axis=0)
                    for t in range(1, 6)], axis=1)
            s6 = jax.lax.dot_general(
                c6, onehot, (((0,), (0,)), ((), ())),
                preferred_element_type=jnp.float32)
            b = None
            for t in range(6):
                w_t = _WC[t, 0] + f * (
                    _WC[t, 1] + f * (_WC[t, 2] + f * (
                        _WC[t, 3] + f * (_WC[t, 4] + f * _WC[t, 5]))))
                s_t = s6[16 * t:16 * (t + 1)]
                b = w_t * s_t if b is None else b + w_t * s_t
            row = jnp.sum(b * kl, axis=0, keepdims=True)
            u_s[pl.ds(d, 1), :] = row
            mm_s[0] = jnp.minimum(mm_s[0], jnp.min(row))
            mm_s[1] = jnp.maximum(mm_s[1], jnp.max(row))

    @pl.when(g >= PH2)
    def _norm():
        d0 = (g - PH2) * NB
        gmin = mm_s[0]
        gmax = mm_s[1]
        out_ref[:, 0, :] = (u_s[pl.ds(d0, NB), :] - gmin) / (
            gmax - gmin + 1e-10)


def kernel(x):
    slices = x.reshape(D, 1, HW)
    M, KL, S = _spatial_consts()

    out = pl.pallas_call(
        _fused_kernel,
        grid=(PH3,),
        in_specs=[
            pl.BlockSpec(memory_space=pltpu.SMEM),
            pl.BlockSpec((TD, 1, HW),
                         lambda g: (jnp.clip(jnp.where(g < PH1, g, g - PH1),
                                             0, D // TD - 1), 0, 0)),
            pl.BlockSpec((GRID[1] * GRID[2], HW), lambda g: (0, 0)),
            pl.BlockSpec((HW, GRID[1] * GRID[2]), lambda g: (0, 0)),
        ],
        out_specs=pl.BlockSpec((NB, 1, HW),
                               lambda g: (jnp.clip(g - PH2, 0,
                                                   D // NB - 1), 0, 0)),
        out_shape=jax.ShapeDtypeStruct((D, 1, HW), jnp.float32),
        scratch_shapes=[
            pltpu.VMEM((NEXT, N_TILES), jnp.float32),
            pltpu.VMEM((D, HW), jnp.float32),
            pltpu.SMEM((2,), jnp.float32),
        ],
    )(M, slices, KL, S)

    return out.reshape(1, 1, D, H, W)
